# baseline (speedup 1.0000x reference)
import functools

import jax
import jax.numpy as jnp
from jax import lax
from jax.experimental import pallas as pl
from jax.experimental.pallas import tpu as pltpu
from jax.experimental.pallas import tpu_sc as plsc

_LP = 1024
_K = 64


def _scores_tc(features2d, W, n_graphs, npg):
    N, C = features2d.shape
    rows = min(40960, N)
    grid = -(-N // rows)

    def body(w_ref, f_ref, o_ref):
        w = w_ref[...]
        nrm = jnp.sqrt(jnp.sum(w * w))
        f = f_ref[...]
        s_row = lax.dot_general(
            w, f, (((1,), (1,)), ((), ())),
            precision=lax.Precision.DEFAULT,
            preferred_element_type=jnp.float32)
        o_ref[...] = (s_row / nrm).reshape(rows)

    return pl.pallas_call(
        body,
        grid=(grid,),
        in_specs=[
            pl.BlockSpec((1, C), lambda g: (0, 0)),
            pl.BlockSpec((rows, C), lambda g: (g, 0)),
        ],
        out_specs=pl.BlockSpec((rows,), lambda g: (g,)),
        out_shape=jax.ShapeDtypeStruct((N,), jnp.float32),
    )(W, features2d)


def _sc_topk_gather(scores_flat, features2d, n_graphs, npg):
    C = features2d.shape[1]
    info = plsc.get_sparse_core_info()
    NC, NS = info.num_cores, info.num_subcores
    NW = NC * NS
    graphs_per_w = -(-n_graphs // NW)
    mesh = plsc.VectorSubcoreMesh(core_axis_name="c", subcore_axis_name="s")

    @functools.partial(
        pl.kernel,
        out_type=[
            jax.ShapeDtypeStruct((n_graphs * _K, C), jnp.float32),
            jax.ShapeDtypeStruct((n_graphs * _K,), jnp.int32),
        ],
        mesh=mesh,
        compiler_params=pltpu.CompilerParams(needs_layout_passes=False),
        scratch_types=[
            pltpu.VMEM((_LP,), jnp.float32),
            pltpu.VMEM((_LP,), jnp.float32),
            pltpu.VMEM((_LP,), jnp.int32),
            pltpu.VMEM((_K + 16,), jnp.int32),
            pltpu.VMEM((_K + 16,), jnp.int32),
            pltpu.VMEM((_K,), jnp.float32),
            pltpu.VMEM((_K,), jnp.int32),
            pltpu.VMEM((_K,), jnp.int32),
            pltpu.VMEM((_K,), jnp.float32),
            pltpu.VMEM((_K, C), jnp.float32),
            pltpu.VMEM((_K, C), jnp.float32),
            pltpu.SemaphoreType.DMA,
            pltpu.SemaphoreType.DMA,
            pltpu.SemaphoreType.DMA,
            pltpu.SemaphoreType.DMA,
        ],
    )
    def k(scores_hbm, feat_hbm, out_hbm, topk_hbm,
          sbufA, sbufB, selbuf, idxbuf, rnkbuf, rsbuf, gidxA, gidxB,
          sigbuf, rowsA, rowsB, ssem, gsem, tsem, rsem):
        sbufs, gidxs, rowss = (sbufA, sbufB), (gidxA, gidxB), (rowsA, rowsB)
        wid = lax.axis_index("s") * NC + lax.axis_index("c")
        iota = lax.iota(jnp.int32, 16)
        minf16 = jnp.full((16,), -jnp.inf, jnp.float32)
        neg16 = jnp.full((16,), -1, jnp.int32)
        big = jnp.full((16,), 16 * _LP, jnp.int32)

        n_full = npg // 16
        tail = npg % 16

        def start_score_copy(g, b):
            pltpu.async_copy(scores_hbm.at[pl.ds(g * npg, npg)],
                             sbufs[b].at[pl.ds(0, npg)], ssem)

        def wait_score_copy(b):
            pltpu.make_async_copy(scores_hbm.at[pl.ds(0, npg)],
                                  sbufs[b].at[pl.ds(0, npg)], ssem).wait()

        def wait_out_stores(b):
            pltpu.make_async_copy(gidxs[b],
                                  topk_hbm.at[pl.ds(0, _K)], tsem).wait()
            pltpu.make_async_copy(rowss[b],
                                  out_hbm.at[pl.ds(0, _K)], rsem).wait()

        def do_graph(g, t):
            b = t & 1
            wait_score_copy(b)
            if t + 1 < graphs_per_w:
                gn = g + NW
                @pl.when(gn < n_graphs)
                def _():
                    start_score_copy(gn, 1 - b)
            if t >= 2:
                wait_out_stores(b)

            if tail:
                v = sbufs[b][pl.ds(n_full * 16, 16)]
                sbufs[b][pl.ds(n_full * 16, 16)] = jnp.where(
                    iota >= tail, -jnp.inf, v)
            for c in range(n_full + (1 if tail else 0), _LP // 16):
                sbufs[b][pl.ds(c * 16, 16)] = minf16

            def chunk_max(cc):
                acc = minf16
                for i in range(16):
                    v = sbufs[b][pl.ds((cc * 16 + i) * 16, 16)]
                    acc = jnp.where(iota == i, jnp.max(v), acc)
                return acc
            cm = [chunk_max(cc) for cc in range(4)]

            def clear_sel(c, _):
                selbuf[pl.ds(c * 16, 16)] = neg16
                return 0
            lax.fori_loop(0, _LP // 16, clear_sel, 0)

            def round_body(r, cms):
                cm0, cm1, cm2, cm3 = cms
                m = jnp.maximum(jnp.maximum(cm0, cm1), jnp.maximum(cm2, cm3))
                gm = jnp.max(m)
                cand = jnp.minimum(
                    jnp.minimum(jnp.where(cm0 == gm, iota, big),
                                jnp.where(cm1 == gm, iota + 16, big)),
                    jnp.minimum(jnp.where(cm2 == gm, iota + 32, big),
                                jnp.where(cm3 == gm, iota + 48, big)))
                cstar = jnp.min(cand)
                v = sbufs[b][pl.ds(cstar * 16, 16)]
                eq = v == gm
                lane = jnp.min(jnp.where(eq, iota, big))
                v2 = jnp.where(iota == lane, -jnp.inf, v)
                sbufs[b][pl.ds(cstar * 16, 16)] = v2
                nmask = jnp.max(jnp.where(eq, minf16, v))
                ngm = plsc.all_reduce_population_count(eq)[0]
                nm = jnp.where(ngm >= 2, gm, nmask)
                sv = selbuf[pl.ds(cstar * 16, 16)]
                selbuf[pl.ds(cstar * 16, 16)] = jnp.where(iota == lane, r, sv)
                rc = rsbuf[pl.ds((r // 16) * 16, 16)]
                rsbuf[pl.ds((r // 16) * 16, 16)] = jnp.where(
                    iota == r % 16, gm, rc)
                ccc = cstar // 16
                cm0 = jnp.where((ccc == 0) & (iota == cstar % 16), nm, cm0)
                cm1 = jnp.where((ccc == 1) & (iota == cstar % 16), nm, cm1)
                cm2 = jnp.where((ccc == 2) & (iota == cstar % 16), nm, cm2)
                cm3 = jnp.where((ccc == 3) & (iota == cstar % 16), nm, cm3)
                return (cm0, cm1, cm2, cm3)
            lax.fori_loop(0, _K, round_body, tuple(cm))

            def compact(c, cnt):
                sv = selbuf[pl.ds(c * 16, 16)]
                msk = sv >= 0
                gi = iota + c * 16
                plsc.store_compressed(idxbuf.at[pl.ds(cnt, 16)], gi, mask=msk)
                plsc.store_compressed(rnkbuf.at[pl.ds(cnt, 16)], sv, mask=msk)
                return cnt + plsc.all_reduce_population_count(msk)[0]
            lax.fori_loop(0, _LP // 16, compact, jnp.int32(0))

            def fetch(cc, _):
                li = idxbuf[pl.ds(cc * 16, 16)]
                rk = rnkbuf[pl.ds(cc * 16, 16)]
                ys = plsc.load_gather(rsbuf, [rk])
                sigbuf[pl.ds(cc * 16, 16)] = 1.0 / (1.0 + jnp.exp(-ys))
                gidxs[b][pl.ds(cc * 16, 16)] = li + g * npg
                return 0
            lax.fori_loop(0, _K // 16, fetch, 0)

            pltpu.async_copy(gidxs[b], topk_hbm.at[pl.ds(g * _K, _K)], tsem)

            pltpu.async_copy(feat_hbm.at[gidxs[b]], rowss[b], gsem).wait()

            def mul_group(rg, _):
                sgv = sigbuf[pl.ds(rg * 16, 16)]
                for lane in range(16):
                    sg = sgv[lane]
                    r_ = rg * 16 + lane
                    for j in range(C // 16):
                        rv = rowss[b][r_, pl.ds(j * 16, 16)]
                        rowss[b][r_, pl.ds(j * 16, 16)] = rv * sg
                return 0
            lax.fori_loop(0, _K // 16, mul_group, 0)

            pltpu.async_copy(rowss[b], out_hbm.at[pl.ds(g * _K, _K)], rsem)

        g0 = wid
        @pl.when(g0 < n_graphs)
        def _():
            start_score_copy(g0, 0)

        for t in range(graphs_per_w):
            g = wid + NW * t
            @pl.when(g < n_graphs)
            def _():
                do_graph(g, t)

        assert n_graphs >= 2 * NW
        wait_out_stores((graphs_per_w - 1) & 1)
        wait_out_stores(graphs_per_w & 1)

    return k(scores_flat, features2d)


def kernel(features_0, batch_num_nodes, W):
    N, C, _ = features_0.shape
    n_graphs = int(batch_num_nodes.shape[0])
    npg = N // n_graphs
    f2d = features_0.reshape(N, C)
    scores = _scores_tc(f2d, W, n_graphs, npg)
    out2d, topk = _sc_topk_gather(scores, f2d, n_graphs, npg)
    return out2d[:, :, None], topk


# speedup vs baseline: 1.0328x; 1.0328x over previous
import functools

import jax
import jax.numpy as jnp
from jax import lax
from jax.experimental import pallas as pl
from jax.experimental.pallas import tpu as pltpu
from jax.experimental.pallas import tpu_sc as plsc

_LP = 1024
_K = 64


def _scores_tc(features2d, W, n_graphs, npg):
    N, C = features2d.shape
    rows = min(20480, N)
    grid = -(-N // rows)

    def body(w_ref, f_ref, o_ref):
        w = w_ref[...]
        nrm = jnp.sqrt(jnp.sum(w * w))
        f = f_ref[...]
        s_row = lax.dot_general(
            w, f, (((1,), (1,)), ((), ())),
            precision=lax.Precision.DEFAULT,
            preferred_element_type=jnp.float32)
        o_ref[...] = (s_row / nrm).reshape(rows)

    return pl.pallas_call(
        body,
        grid=(grid,),
        in_specs=[
            pl.BlockSpec((1, C), lambda g: (0, 0)),
            pl.BlockSpec((rows, C), lambda g: (g, 0)),
        ],
        out_specs=pl.BlockSpec((rows,), lambda g: (g,)),
        out_shape=jax.ShapeDtypeStruct((N,), jnp.float32),
    )(W, features2d)


def _sc_topk_gather(scores_flat, features2d, n_graphs, npg):
    C = features2d.shape[1]
    info = plsc.get_sparse_core_info()
    NC, NS = info.num_cores, info.num_subcores
    NW = NC * NS
    graphs_per_w = -(-n_graphs // NW)
    mesh = plsc.VectorSubcoreMesh(core_axis_name="c", subcore_axis_name="s")

    @functools.partial(
        pl.kernel,
        out_type=[
            jax.ShapeDtypeStruct((n_graphs * _K, C), jnp.float32),
            jax.ShapeDtypeStruct((n_graphs * _K,), jnp.int32),
        ],
        mesh=mesh,
        compiler_params=pltpu.CompilerParams(needs_layout_passes=False),
        scratch_types=[
            pltpu.VMEM((_LP,), jnp.float32),
            pltpu.VMEM((_LP,), jnp.float32),
            pltpu.VMEM((_LP,), jnp.int32),
            pltpu.VMEM((_K + 16,), jnp.int32),
            pltpu.VMEM((_K + 16,), jnp.int32),
            pltpu.VMEM((_K,), jnp.float32),
            pltpu.VMEM((_K,), jnp.int32),
            pltpu.VMEM((_K,), jnp.int32),
            pltpu.VMEM((_K,), jnp.float32),
            pltpu.VMEM((_K, C), jnp.float32),
            pltpu.VMEM((_K, C), jnp.float32),
            pltpu.SemaphoreType.DMA,
            pltpu.SemaphoreType.DMA,
            pltpu.SemaphoreType.DMA,
            pltpu.SemaphoreType.DMA,
        ],
    )
    def k(scores_hbm, feat_hbm, out_hbm, topk_hbm,
          sbufA, sbufB, selbuf, idxbuf, rnkbuf, rsbuf, gidxA, gidxB,
          sigbuf, rowsA, rowsB, ssem, gsem, tsem, rsem):
        sbufs, gidxs, rowss = (sbufA, sbufB), (gidxA, gidxB), (rowsA, rowsB)
        wid = lax.axis_index("s") * NC + lax.axis_index("c")
        iota = lax.iota(jnp.int32, 16)
        minf16 = jnp.full((16,), -jnp.inf, jnp.float32)
        neg16 = jnp.full((16,), -1, jnp.int32)
        big = jnp.full((16,), 16 * _LP, jnp.int32)

        n_full = npg // 16
        tail = npg % 16

        def start_score_copy(g, b):
            pltpu.async_copy(scores_hbm.at[pl.ds(g * npg, npg)],
                             sbufs[b].at[pl.ds(0, npg)], ssem)

        def wait_score_copy(b):
            pltpu.make_async_copy(scores_hbm.at[pl.ds(0, npg)],
                                  sbufs[b].at[pl.ds(0, npg)], ssem).wait()

        def wait_out_stores(b):
            pltpu.make_async_copy(gidxs[b],
                                  topk_hbm.at[pl.ds(0, _K)], tsem).wait()
            pltpu.make_async_copy(rowss[b],
                                  out_hbm.at[pl.ds(0, _K)], rsem).wait()

        def do_graph(g, t):
            b = t & 1
            wait_score_copy(b)
            if t + 1 < graphs_per_w:
                gn = g + NW
                @pl.when(gn < n_graphs)
                def _():
                    start_score_copy(gn, 1 - b)
            if t >= 2:
                wait_out_stores(b)

            if tail:
                v = sbufs[b][pl.ds(n_full * 16, 16)]
                sbufs[b][pl.ds(n_full * 16, 16)] = jnp.where(
                    iota >= tail, -jnp.inf, v)
            for c in range(n_full + (1 if tail else 0), _LP // 16):
                sbufs[b][pl.ds(c * 16, 16)] = minf16

            def chunk_max(cc):
                acc = minf16
                for i in range(16):
                    v = sbufs[b][pl.ds((cc * 16 + i) * 16, 16)]
                    acc = jnp.where(iota == i, jnp.max(v), acc)
                return acc
            cm = [chunk_max(cc) for cc in range(4)]

            def clear_sel(c, _):
                selbuf[pl.ds(c * 16, 16)] = neg16
                return 0
            lax.fori_loop(0, _LP // 16, clear_sel, 0)

            def round_body(r, cms):
                cm0, cm1, cm2, cm3 = cms
                m = jnp.maximum(jnp.maximum(cm0, cm1), jnp.maximum(cm2, cm3))
                gm = jnp.max(m)
                cand = jnp.minimum(
                    jnp.minimum(jnp.where(cm0 == gm, iota, big),
                                jnp.where(cm1 == gm, iota + 16, big)),
                    jnp.minimum(jnp.where(cm2 == gm, iota + 32, big),
                                jnp.where(cm3 == gm, iota + 48, big)))
                cstar = jnp.min(cand)
                v = sbufs[b][pl.ds(cstar * 16, 16)]
                eq = v == gm
                lane = jnp.min(jnp.where(eq, iota, big))
                v2 = jnp.where(iota == lane, -jnp.inf, v)
                sbufs[b][pl.ds(cstar * 16, 16)] = v2
                nmask = jnp.max(jnp.where(eq, minf16, v))
                ngm = plsc.all_reduce_population_count(eq)[0]
                nm = jnp.where(ngm >= 2, gm, nmask)
                sv = selbuf[pl.ds(cstar * 16, 16)]
                selbuf[pl.ds(cstar * 16, 16)] = jnp.where(iota == lane, r, sv)
                rc = rsbuf[pl.ds((r // 16) * 16, 16)]
                rsbuf[pl.ds((r // 16) * 16, 16)] = jnp.where(
                    iota == r % 16, gm, rc)
                ccc = cstar // 16
                cm0 = jnp.where((ccc == 0) & (iota == cstar % 16), nm, cm0)
                cm1 = jnp.where((ccc == 1) & (iota == cstar % 16), nm, cm1)
                cm2 = jnp.where((ccc == 2) & (iota == cstar % 16), nm, cm2)
                cm3 = jnp.where((ccc == 3) & (iota == cstar % 16), nm, cm3)
                return (cm0, cm1, cm2, cm3)
            lax.fori_loop(0, _K, round_body, tuple(cm))

            def compact(c, cnt):
                sv = selbuf[pl.ds(c * 16, 16)]
                msk = sv >= 0
                gi = iota + c * 16
                plsc.store_compressed(idxbuf.at[pl.ds(cnt, 16)], gi, mask=msk)
                plsc.store_compressed(rnkbuf.at[pl.ds(cnt, 16)], sv, mask=msk)
                return cnt + plsc.all_reduce_population_count(msk)[0]
            lax.fori_loop(0, _LP // 16, compact, jnp.int32(0))

            def fetch(cc, _):
                li = idxbuf[pl.ds(cc * 16, 16)]
                rk = rnkbuf[pl.ds(cc * 16, 16)]
                ys = plsc.load_gather(rsbuf, [rk])
                sigbuf[pl.ds(cc * 16, 16)] = 1.0 / (1.0 + jnp.exp(-ys))
                gidxs[b][pl.ds(cc * 16, 16)] = li + g * npg
                return 0
            lax.fori_loop(0, _K // 16, fetch, 0)

            pltpu.async_copy(gidxs[b], topk_hbm.at[pl.ds(g * _K, _K)], tsem)

            pltpu.async_copy(feat_hbm.at[gidxs[b]], rowss[b], gsem).wait()

            def mul_group(rg, _):
                sgv = sigbuf[pl.ds(rg * 16, 16)]
                for lane in range(16):
                    sg = sgv[lane]
                    r_ = rg * 16 + lane
                    for j in range(C // 16):
                        rv = rowss[b][r_, pl.ds(j * 16, 16)]
                        rowss[b][r_, pl.ds(j * 16, 16)] = rv * sg
                return 0
            lax.fori_loop(0, _K // 16, mul_group, 0)

            pltpu.async_copy(rowss[b], out_hbm.at[pl.ds(g * _K, _K)], rsem)

        g0 = wid
        @pl.when(g0 < n_graphs)
        def _():
            start_score_copy(g0, 0)

        for t in range(graphs_per_w):
            g = wid + NW * t
            @pl.when(g < n_graphs)
            def _():
                do_graph(g, t)

        assert n_graphs >= 2 * NW
        wait_out_stores((graphs_per_w - 1) & 1)
        wait_out_stores(graphs_per_w & 1)

    return k(scores_flat, features2d)


def kernel(features_0, batch_num_nodes, W):
    N, C, _ = features_0.shape
    n_graphs = int(batch_num_nodes.shape[0])
    npg = N // n_graphs
    f2d = features_0.reshape(N, C)
    scores = _scores_tc(f2d, W, n_graphs, npg)
    out2d, topk = _sc_topk_gather(scores, f2d, n_graphs, npg)
    return out2d[:, :, None], topk


# speedup vs baseline: 1.0475x; 1.0142x over previous
import functools

import jax
import jax.numpy as jnp
from jax import lax
from jax.experimental import pallas as pl
from jax.experimental.pallas import tpu as pltpu
from jax.experimental.pallas import tpu_sc as plsc

_LP = 1024
_K = 64


def _scores_tc(features2d, W, n_graphs, npg):
    N, C = features2d.shape
    rows = min(20480, N)
    grid = -(-N // rows)

    def body(w_ref, f_ref, o_ref):
        w = w_ref[...]
        nrm = jnp.sqrt(jnp.sum(w * w))
        f = f_ref[...]
        s_row = lax.dot_general(
            w, f, (((1,), (1,)), ((), ())),
            precision=lax.Precision.DEFAULT,
            preferred_element_type=jnp.float32)
        o_ref[...] = (s_row / nrm).reshape(rows)

    return pl.pallas_call(
        body,
        grid=(grid,),
        in_specs=[
            pl.BlockSpec((1, C), lambda g: (0, 0)),
            pl.BlockSpec((rows, C), lambda g: (g, 0)),
        ],
        out_specs=pl.BlockSpec((rows,), lambda g: (g,)),
        out_shape=jax.ShapeDtypeStruct((N,), jnp.float32),
    )(W, features2d)


def _sc_topk_gather(scores_flat, features2d, n_graphs, npg):
    C = features2d.shape[1]
    info = plsc.get_sparse_core_info()
    NC, NS = info.num_cores, info.num_subcores
    NW = NC * NS
    graphs_per_w = -(-n_graphs // NW)
    mesh = plsc.VectorSubcoreMesh(core_axis_name="c", subcore_axis_name="s")

    @functools.partial(
        pl.kernel,
        out_type=[
            jax.ShapeDtypeStruct((n_graphs * _K, C), jnp.float32),
            jax.ShapeDtypeStruct((n_graphs * _K,), jnp.int32),
        ],
        mesh=mesh,
        compiler_params=pltpu.CompilerParams(needs_layout_passes=False),
        scratch_types=[
            pltpu.VMEM((_LP,), jnp.float32),
            pltpu.VMEM((_LP,), jnp.float32),
            pltpu.VMEM((_LP,), jnp.int32),
            pltpu.VMEM((_K + 16,), jnp.int32),
            pltpu.VMEM((_K + 16,), jnp.int32),
            pltpu.VMEM((_K,), jnp.float32),
            pltpu.VMEM((_K,), jnp.int32),
            pltpu.VMEM((_K,), jnp.int32),
            pltpu.VMEM((_K,), jnp.float32),
            pltpu.VMEM((_K, C), jnp.float32),
            pltpu.VMEM((_K, C), jnp.float32),
            pltpu.SemaphoreType.DMA,
            pltpu.SemaphoreType.DMA,
            pltpu.SemaphoreType.DMA,
            pltpu.SemaphoreType.DMA,
        ],
    )
    def k(scores_hbm, feat_hbm, out_hbm, topk_hbm,
          sbufA, sbufB, selbuf, idxbuf, rnkbuf, rsbuf, gidxA, gidxB,
          sigbuf, rowsA, rowsB, ssem, gsem, tsem, rsem):
        sbufs, gidxs, rowss = (sbufA, sbufB), (gidxA, gidxB), (rowsA, rowsB)
        wid = lax.axis_index("s") * NC + lax.axis_index("c")
        iota = lax.iota(jnp.int32, 16)
        minf16 = jnp.full((16,), -jnp.inf, jnp.float32)
        neg16 = jnp.full((16,), -1, jnp.int32)
        big = jnp.full((16,), 16 * _LP, jnp.int32)

        n_full = npg // 16
        tail = npg % 16

        def start_score_copy(g, b):
            pltpu.async_copy(scores_hbm.at[pl.ds(g * npg, npg)],
                             sbufs[b].at[pl.ds(0, npg)], ssem)

        def wait_score_copy(b):
            pltpu.make_async_copy(scores_hbm.at[pl.ds(0, npg)],
                                  sbufs[b].at[pl.ds(0, npg)], ssem).wait()

        def wait_out_stores(b):
            pltpu.make_async_copy(gidxs[b],
                                  topk_hbm.at[pl.ds(0, _K)], tsem).wait()
            pltpu.make_async_copy(rowss[b],
                                  out_hbm.at[pl.ds(0, _K)], rsem).wait()

        def do_graph(g, t):
            b = t & 1
            def clear_sel(c, _):
                selbuf[pl.ds(c * 16, 16)] = neg16
                return 0
            lax.fori_loop(0, _LP // 16, clear_sel, 0)
            if t >= 2:
                wait_out_stores(b)
            wait_score_copy(b)
            if t + 1 < graphs_per_w:
                gn = g + NW
                @pl.when(gn < n_graphs)
                def _():
                    start_score_copy(gn, 1 - b)

            if tail:
                v = sbufs[b][pl.ds(n_full * 16, 16)]
                sbufs[b][pl.ds(n_full * 16, 16)] = jnp.where(
                    iota >= tail, -jnp.inf, v)
            for c in range(n_full + (1 if tail else 0), _LP // 16):
                sbufs[b][pl.ds(c * 16, 16)] = minf16

            def chunk_max(cc):
                acc = minf16
                for i in range(16):
                    v = sbufs[b][pl.ds((cc * 16 + i) * 16, 16)]
                    acc = jnp.where(iota == i, jnp.max(v), acc)
                return acc
            cm = [chunk_max(cc) for cc in range(4)]

            def round_body(r, cms):
                cm0, cm1, cm2, cm3 = cms
                m = jnp.maximum(jnp.maximum(cm0, cm1), jnp.maximum(cm2, cm3))
                gm = jnp.max(m)
                cand = jnp.minimum(
                    jnp.minimum(jnp.where(cm0 == gm, iota, big),
                                jnp.where(cm1 == gm, iota + 16, big)),
                    jnp.minimum(jnp.where(cm2 == gm, iota + 32, big),
                                jnp.where(cm3 == gm, iota + 48, big)))
                cstar = jnp.min(cand)
                v = sbufs[b][pl.ds(cstar * 16, 16)]
                eq = v == gm
                lane = jnp.min(jnp.where(eq, iota, big))
                v2 = jnp.where(iota == lane, -jnp.inf, v)
                sbufs[b][pl.ds(cstar * 16, 16)] = v2
                nmask = jnp.max(jnp.where(eq, minf16, v))
                ngm = plsc.all_reduce_population_count(eq)[0]
                nm = jnp.where(ngm >= 2, gm, nmask)
                sv = selbuf[pl.ds(cstar * 16, 16)]
                selbuf[pl.ds(cstar * 16, 16)] = jnp.where(iota == lane, r, sv)
                rc = rsbuf[pl.ds((r // 16) * 16, 16)]
                rsbuf[pl.ds((r // 16) * 16, 16)] = jnp.where(
                    iota == r % 16, gm, rc)
                ccc = cstar // 16
                cm0 = jnp.where((ccc == 0) & (iota == cstar % 16), nm, cm0)
                cm1 = jnp.where((ccc == 1) & (iota == cstar % 16), nm, cm1)
                cm2 = jnp.where((ccc == 2) & (iota == cstar % 16), nm, cm2)
                cm3 = jnp.where((ccc == 3) & (iota == cstar % 16), nm, cm3)
                return (cm0, cm1, cm2, cm3)
            lax.fori_loop(0, _K, round_body, tuple(cm))

            def compact(c, cnt):
                sv = selbuf[pl.ds(c * 16, 16)]
                msk = sv >= 0
                gi = iota + c * 16
                plsc.store_compressed(idxbuf.at[pl.ds(cnt, 16)], gi, mask=msk)
                plsc.store_compressed(rnkbuf.at[pl.ds(cnt, 16)], sv, mask=msk)
                return cnt + plsc.all_reduce_population_count(msk)[0]
            lax.fori_loop(0, _LP // 16, compact, jnp.int32(0))

            def fetch(cc, _):
                li = idxbuf[pl.ds(cc * 16, 16)]
                rk = rnkbuf[pl.ds(cc * 16, 16)]
                ys = plsc.load_gather(rsbuf, [rk])
                sigbuf[pl.ds(cc * 16, 16)] = 1.0 / (1.0 + jnp.exp(-ys))
                gidxs[b][pl.ds(cc * 16, 16)] = li + g * npg
                return 0
            lax.fori_loop(0, _K // 16, fetch, 0)

            pltpu.async_copy(gidxs[b], topk_hbm.at[pl.ds(g * _K, _K)], tsem)

            pltpu.async_copy(feat_hbm.at[gidxs[b]], rowss[b], gsem).wait()

            def mul_group(rg, _):
                sgv = sigbuf[pl.ds(rg * 16, 16)]
                for lane in range(16):
                    sg = sgv[lane]
                    r_ = rg * 16 + lane
                    for j in range(C // 16):
                        rv = rowss[b][r_, pl.ds(j * 16, 16)]
                        rowss[b][r_, pl.ds(j * 16, 16)] = rv * sg
                return 0
            lax.fori_loop(0, _K // 16, mul_group, 0)

            pltpu.async_copy(rowss[b], out_hbm.at[pl.ds(g * _K, _K)], rsem)

        g0 = wid
        @pl.when(g0 < n_graphs)
        def _():
            start_score_copy(g0, 0)

        for t in range(graphs_per_w):
            g = wid + NW * t
            @pl.when(g < n_graphs)
            def _():
                do_graph(g, t)

        assert n_graphs >= 2 * NW
        wait_out_stores((graphs_per_w - 1) & 1)
        wait_out_stores(graphs_per_w & 1)

    return k(scores_flat, features2d)


def kernel(features_0, batch_num_nodes, W):
    N, C, _ = features_0.shape
    n_graphs = int(batch_num_nodes.shape[0])
    npg = N // n_graphs
    f2d = features_0.reshape(N, C)
    scores = _scores_tc(f2d, W, n_graphs, npg)
    out2d, topk = _sc_topk_gather(scores, f2d, n_graphs, npg)
    return out2d[:, :, None], topk
